# baseline (speedup 1.0000x reference)
import functools

import jax
from jax import lax
from jax.experimental import pallas as pl
from jax.experimental.pallas import tpu as pltpu
from jax.experimental.pallas import tpu_sc as plsc

_CHUNK_ROWS = 32
_NBUF = 3


def _make_sc_copy(seq_len, d_model, dtype):
    info = plsc.get_sparse_core_info()
    nc, ns = info.num_cores, info.num_subcores
    nw = nc * ns
    rows_per = seq_len // nw
    nchunks = rows_per // _CHUNK_ROWS
    mesh = plsc.VectorSubcoreMesh(core_axis_name="c", subcore_axis_name="s")

    scratch = [pltpu.VMEM((_CHUNK_ROWS, d_model), dtype)] * _NBUF
    scratch += [pltpu.SemaphoreType.DMA] * (2 * _NBUF)

    @functools.partial(
        pl.kernel,
        mesh=mesh,
        out_type=jax.ShapeDtypeStruct((seq_len, d_model), dtype),
        scratch_types=scratch,
    )
    def sc_copy(table_hbm, out_hbm, *scr):
        bufs = scr[:_NBUF]
        gsem = scr[_NBUF:2 * _NBUF]
        ssem = scr[2 * _NBUF:]
        wid = lax.axis_index("c") * ns + lax.axis_index("s")
        base = wid * rows_per
        scat = [None] * _NBUF
        for i in range(nchunks):
            b = i % _NBUF
            lo = base + i * _CHUNK_ROWS
            if scat[b] is not None:
                scat[b].wait()
            gath = pltpu.async_copy(
                table_hbm.at[pl.ds(lo, _CHUNK_ROWS)], bufs[b], gsem[b]
            )
            gath.wait()
            scat[b] = pltpu.async_copy(
                bufs[b], out_hbm.at[pl.ds(lo, _CHUNK_ROWS)], ssem[b]
            )
        for b in range(_NBUF):
            if scat[b] is not None:
                scat[b].wait()

    return sc_copy


def kernel(x, pos_table):
    seq_len = x.shape[1]
    d_model = pos_table.shape[1]
    table = pos_table[:seq_len]
    out = _make_sc_copy(seq_len, d_model, pos_table.dtype)(table)
    return out[None]


# speedup vs baseline: 1.0814x; 1.0814x over previous
import functools

import jax
from jax import lax
from jax.experimental import pallas as pl
from jax.experimental.pallas import tpu as pltpu
from jax.experimental.pallas import tpu_sc as plsc

_CHUNK_ROWS = 32


def _make_sc_copy(seq_len, d_model, dtype):
    info = plsc.get_sparse_core_info()
    nc, ns = info.num_cores, info.num_subcores
    nw = nc * ns
    rows_per = seq_len // nw
    npairs = rows_per // (2 * _CHUNK_ROWS)
    mesh = plsc.VectorSubcoreMesh(core_axis_name="c", subcore_axis_name="s")

    scratch = [
        pltpu.VMEM((_CHUNK_ROWS, d_model), dtype),
        pltpu.VMEM((_CHUNK_ROWS, d_model), dtype),
        pltpu.VMEM_SHARED((2 * ns, _CHUNK_ROWS, d_model), dtype),
    ]
    scratch += [pltpu.SemaphoreType.DMA] * 8

    @functools.partial(
        pl.kernel,
        mesh=mesh,
        out_type=jax.ShapeDtypeStruct((seq_len, d_model), dtype),
        scratch_types=scratch,
    )
    def sc_copy(table_hbm, out_hbm, tb0, tb1, shared, *sems):
        tg = sems[0:2]
        ts = sems[2:4]
        sg = sems[4:6]
        ss = sems[6:8]
        sid = lax.axis_index("s")
        wid = lax.axis_index("c") * ns + sid
        base = wid * rows_per
        tbufs = (tb0, tb1)
        tscat = [None, None]
        sscat = [None, None]
        for p in range(npairs):
            b = p % 2
            lo_t = base + (2 * p) * _CHUNK_ROWS
            lo_s = base + (2 * p + 1) * _CHUNK_ROWS
            sbuf = shared.at[2 * sid + b]
            if tscat[b] is not None:
                tscat[b].wait()
            gt = pltpu.async_copy(
                table_hbm.at[pl.ds(lo_t, _CHUNK_ROWS)], tbufs[b], tg[b]
            )
            if sscat[b] is not None:
                sscat[b].wait()
            gs = pltpu.async_copy(
                table_hbm.at[pl.ds(lo_s, _CHUNK_ROWS)], sbuf, sg[b]
            )
            gt.wait()
            tscat[b] = pltpu.async_copy(
                tbufs[b], out_hbm.at[pl.ds(lo_t, _CHUNK_ROWS)], ts[b]
            )
            gs.wait()
            sscat[b] = pltpu.async_copy(
                sbuf, out_hbm.at[pl.ds(lo_s, _CHUNK_ROWS)], ss[b]
            )
        for b in (0, 1):
            if tscat[b] is not None:
                tscat[b].wait()
            if sscat[b] is not None:
                sscat[b].wait()

    return sc_copy


def kernel(x, pos_table):
    seq_len = x.shape[1]
    d_model = pos_table.shape[1]
    table = pos_table[:seq_len]
    out = _make_sc_copy(seq_len, d_model, pos_table.dtype)(table)
    return out[None]
